# speedup vs baseline: 1.2591x; 1.2591x over previous
import functools

import jax
import jax.numpy as jnp
import numpy as np
from jax.experimental import pallas as pl
from jax.experimental.pallas import tpu as pltpu

L = 3
H = 1024
D_IN = 1024
N_TOK = 32768
DECAY = 0.9
K = int(0.1 * H)
C = np.float32(1.0 - DECAY)

BT = 512
NSTEPS = N_TOK // BT

_BIG_I = np.int32(1 << 30)
_INF = np.float32(np.inf)


def _dotT(x, w):
    return jax.lax.dot_general(
        x, w, (((1,), (1,)), ((), ())), preferred_element_type=jnp.float32)


def _fwd_select_kernel(x_ref, w0_ref, w1_ref, w2_ref, w3_ref, b_ref,
                       wu_ref, ma_ref, feats_ref, keep_ref, nb_ref,
                       sacc_ref):
    i = pl.program_id(0)

    @pl.when(i == 0)
    def _():
        sacc_ref[...] = jnp.zeros_like(sacc_ref)

    x = x_ref[...]
    h1 = jax.nn.relu(_dotT(x, w0_ref[...]) + b_ref[0:1, :])
    h2 = jax.nn.relu(_dotT(h1, w1_ref[...]) + b_ref[1:2, :])
    h3 = jax.nn.relu(_dotT(h2, w2_ref[...]) + b_ref[2:3, :])
    sacc_ref[0:1, :] += jnp.sum(h1, axis=0, keepdims=True)
    sacc_ref[1:2, :] += jnp.sum(h2, axis=0, keepdims=True)
    sacc_ref[2:3, :] += jnp.sum(h3, axis=0, keepdims=True)

    @pl.when(i == NSTEPS - 1)
    def _():
        inv_h = np.float32(1.0 / H)
        inv_n = np.float32(1.0 / N_TOK)
        mean8 = sacc_ref[...] * inv_n
        owm1 = jnp.sum(jnp.abs(w1_ref[...]), axis=0, keepdims=True) * inv_h
        owm2 = jnp.sum(jnp.abs(w2_ref[...]), axis=0, keepdims=True) * inv_h
        owm3 = jnp.sum(jnp.abs(w3_ref[...]), axis=0, keepdims=True) * inv_h
        u1 = C * (owm1 * mean8[0:1, :])
        u2 = C * (owm2 * mean8[1:2, :])
        u3 = C * (owm3 * mean8[2:3, :])
        pad = jnp.zeros((5, H), jnp.float32)
        u8 = jnp.concatenate([u1, u2, u3, pad], axis=0)

        lane = jax.lax.broadcasted_iota(jnp.int32, (8, H), 1)
        klane = jax.lax.broadcasted_iota(jnp.int32, (8, 128), 1)

        def body(k, carry):
            u, facc = carry
            m = jnp.min(u, axis=1, keepdims=True)
            idx = jnp.min(jnp.where(u == m, lane, _BIG_I), axis=1,
                          keepdims=True)
            facc = facc + jnp.where(klane == k, idx, 0)
            u = jnp.where(lane == idx, _INF, u)
            return u, facc

        u8f, facc = jax.lax.fori_loop(
            0, K, body, (u8, jnp.zeros((8, 128), jnp.int32)))
        keep = (u8f != _INF).astype(jnp.float32)
        wu_ref[...] = jnp.where(u8f == _INF, 0.0, u8f)
        ma_ref[...] = (C * mean8) * keep
        feats_ref[...] = facc
        keep_ref[...] = keep
        nb_ref[...] = b_ref[...] * jnp.concatenate(
            [keep[0:3, :], jnp.ones((5, H), jnp.float32)], axis=0)


def _weights_kernel(feats_ref, keep_ref, rand_ref,
                    w0_ref, w1_ref, w2_ref, w3_ref,
                    nw0_ref, nw1_ref, nw2_ref, nw3_ref):
    nw0_ref[...] = w0_ref[...]
    nw1_ref[...] = w1_ref[...] * keep_ref[0:1, :]
    nw2_ref[...] = w2_ref[...] * keep_ref[1:2, :]
    nw3_ref[...] = w3_ref[...] * keep_ref[2:3, :]

    for layer, ref in ((0, nw0_ref), (1, nw1_ref), (2, nw2_ref)):
        def body(k, _, ref=ref, layer=layer):
            idx = feats_ref[layer, k]
            ref[pl.ds(idx, 1), :] = rand_ref[pl.ds(layer * K + k, 1), :]
            return 0

        jax.lax.fori_loop(0, K, body, 0)


def _replacement_rows():
    reinit_key = jax.random.key(42)
    rows = [jax.random.normal(jax.random.fold_in(reinit_key, i), (K, H),
                              jnp.float32) * 0.02 for i in range(L)]
    rand = jnp.concatenate(rows, axis=0)
    npad = 8 * ((L * K + 7) // 8) - L * K
    return jnp.concatenate([rand, jnp.zeros((npad, H), jnp.float32)], axis=0)


def kernel(input, W0, W1, W2, W3, b0, b1, b2, b3):
    x = input
    zrow = jnp.zeros((H,), jnp.float32)
    bmat = jnp.stack([b0, b1, b2, b3, zrow, zrow, zrow, zrow])

    res_spec = pl.BlockSpec((H, H), lambda i: (0, 0))
    row8_spec = pl.BlockSpec((8, H), lambda i: (0, 0))
    wu8, ma8, featsp, keep8, nb8 = pl.pallas_call(
        _fwd_select_kernel,
        grid=(NSTEPS,),
        in_specs=[
            pl.BlockSpec((BT, D_IN), lambda i: (i, 0)),
            res_spec, res_spec, res_spec, res_spec,
            row8_spec,
        ],
        out_specs=[row8_spec, row8_spec,
                   pl.BlockSpec((8, 128), lambda i: (0, 0)),
                   row8_spec, row8_spec],
        out_shape=[
            jax.ShapeDtypeStruct((8, H), jnp.float32),
            jax.ShapeDtypeStruct((8, H), jnp.float32),
            jax.ShapeDtypeStruct((8, 128), jnp.int32),
            jax.ShapeDtypeStruct((8, H), jnp.float32),
            jax.ShapeDtypeStruct((8, H), jnp.float32),
        ],
        scratch_shapes=[pltpu.VMEM((8, H), jnp.float32)],
    )(x, W0, W1, W2, W3, bmat)

    rand = _replacement_rows()
    nw0, nw1, nw2, nw3 = pl.pallas_call(
        _weights_kernel,
        in_specs=[
            pl.BlockSpec(memory_space=pltpu.SMEM),
            pl.BlockSpec((8, H), lambda: (0, 0)),
            pl.BlockSpec(rand.shape, lambda: (0, 0)),
            pl.BlockSpec((H, H), lambda: (0, 0)),
            pl.BlockSpec((H, H), lambda: (0, 0)),
            pl.BlockSpec((H, H), lambda: (0, 0)),
            pl.BlockSpec((H, H), lambda: (0, 0)),
        ],
        out_specs=[pl.BlockSpec((H, H), lambda: (0, 0))] * 4,
        out_shape=[jax.ShapeDtypeStruct((H, H), jnp.float32)] * 4,
    )(featsp, keep8, rand, W0, W1, W2, W3)

    weight_util = wu8[:L]
    mean_act = ma8[:L]
    feats = featsp[:L, :K]
    newb0 = nb8[0]
    newb1 = nb8[1]
    newb2 = nb8[2]
    newb3 = nb8[3]
    return (weight_util, mean_act, feats,
            nw0, nw1, nw2, nw3, newb0, newb1, newb2, newb3)


# speedup vs baseline: 1.3153x; 1.0446x over previous
import functools

import jax
import jax.numpy as jnp
import numpy as np
from jax.experimental import pallas as pl
from jax.experimental.pallas import tpu as pltpu

L = 3
H = 1024
D_IN = 1024
N_TOK = 32768
DECAY = 0.9
K = int(0.1 * H)
C = np.float32(1.0 - DECAY)

BT = 1024
NSTEPS = N_TOK // BT

_BIG_I = np.int32(1 << 30)
_INF = np.float32(np.inf)


def _dotT(x, w):
    return jax.lax.dot_general(
        x, w, (((1,), (1,)), ((), ())), preferred_element_type=jnp.float32)


def _fwd_select_kernel(x_ref, w0_ref, w1_ref, w2_ref, w3_ref, b_ref,
                       wu_ref, ma_ref, feats_ref, keep_ref, nb_ref,
                       sacc_ref):
    i = pl.program_id(0)

    @pl.when(i == 0)
    def _():
        sacc_ref[...] = jnp.zeros_like(sacc_ref)

    x = x_ref[...]
    h1 = jax.nn.relu(_dotT(x, w0_ref[...]) + b_ref[0:1, :])
    h2 = jax.nn.relu(_dotT(h1, w1_ref[...]) + b_ref[1:2, :])
    h3 = jax.nn.relu(_dotT(h2, w2_ref[...]) + b_ref[2:3, :])
    sacc_ref[0:1, :] += jnp.sum(h1, axis=0, keepdims=True)
    sacc_ref[1:2, :] += jnp.sum(h2, axis=0, keepdims=True)
    sacc_ref[2:3, :] += jnp.sum(h3, axis=0, keepdims=True)

    @pl.when(i == NSTEPS - 1)
    def _():
        inv_h = np.float32(1.0 / H)
        inv_n = np.float32(1.0 / N_TOK)
        mean8 = sacc_ref[...] * inv_n
        owm1 = jnp.sum(jnp.abs(w1_ref[...]), axis=0, keepdims=True) * inv_h
        owm2 = jnp.sum(jnp.abs(w2_ref[...]), axis=0, keepdims=True) * inv_h
        owm3 = jnp.sum(jnp.abs(w3_ref[...]), axis=0, keepdims=True) * inv_h
        u1 = C * (owm1 * mean8[0:1, :])
        u2 = C * (owm2 * mean8[1:2, :])
        u3 = C * (owm3 * mean8[2:3, :])
        pad = jnp.zeros((5, H), jnp.float32)
        u8 = jnp.concatenate([u1, u2, u3, pad], axis=0)

        lane = jax.lax.broadcasted_iota(jnp.int32, (8, H), 1)
        klane = jax.lax.broadcasted_iota(jnp.int32, (8, 128), 1)

        def body(k, carry):
            u, facc = carry
            m = jnp.min(u, axis=1, keepdims=True)
            idx = jnp.min(jnp.where(u == m, lane, _BIG_I), axis=1,
                          keepdims=True)
            facc = facc + jnp.where(klane == k, idx, 0)
            u = jnp.where(lane == idx, _INF, u)
            return u, facc

        u8f, facc = jax.lax.fori_loop(
            0, K, body, (u8, jnp.zeros((8, 128), jnp.int32)))
        keep = (u8f != _INF).astype(jnp.float32)
        wu_ref[...] = jnp.where(u8f == _INF, 0.0, u8f)
        ma_ref[...] = (C * mean8) * keep
        feats_ref[...] = facc
        keep_ref[...] = keep
        nb_ref[...] = b_ref[...] * jnp.concatenate(
            [keep[0:3, :], jnp.ones((5, H), jnp.float32)], axis=0)


def _weights_kernel(feats_ref, keep_ref, rand_ref,
                    w0_ref, w1_ref, w2_ref, w3_ref,
                    nw0_ref, nw1_ref, nw2_ref, nw3_ref):
    nw0_ref[...] = w0_ref[...]
    nw1_ref[...] = w1_ref[...] * keep_ref[0:1, :]
    nw2_ref[...] = w2_ref[...] * keep_ref[1:2, :]
    nw3_ref[...] = w3_ref[...] * keep_ref[2:3, :]

    for layer, ref in ((0, nw0_ref), (1, nw1_ref), (2, nw2_ref)):
        def body(k, _, ref=ref, layer=layer):
            idx = feats_ref[layer, k]
            ref[pl.ds(idx, 1), :] = rand_ref[pl.ds(layer * K + k, 1), :]
            return 0

        jax.lax.fori_loop(0, K, body, 0)


def _replacement_rows():
    reinit_key = jax.random.key(42)
    rows = [jax.random.normal(jax.random.fold_in(reinit_key, i), (K, H),
                              jnp.float32) * 0.02 for i in range(L)]
    rand = jnp.concatenate(rows, axis=0)
    npad = 8 * ((L * K + 7) // 8) - L * K
    return jnp.concatenate([rand, jnp.zeros((npad, H), jnp.float32)], axis=0)


def kernel(input, W0, W1, W2, W3, b0, b1, b2, b3):
    x = input
    zrow = jnp.zeros((H,), jnp.float32)
    bmat = jnp.stack([b0, b1, b2, b3, zrow, zrow, zrow, zrow])

    res_spec = pl.BlockSpec((H, H), lambda i: (0, 0))
    row8_spec = pl.BlockSpec((8, H), lambda i: (0, 0))
    wu8, ma8, featsp, keep8, nb8 = pl.pallas_call(
        _fwd_select_kernel,
        grid=(NSTEPS,),
        in_specs=[
            pl.BlockSpec((BT, D_IN), lambda i: (i, 0)),
            res_spec, res_spec, res_spec, res_spec,
            row8_spec,
        ],
        out_specs=[row8_spec, row8_spec,
                   pl.BlockSpec((8, 128), lambda i: (0, 0)),
                   row8_spec, row8_spec],
        out_shape=[
            jax.ShapeDtypeStruct((8, H), jnp.float32),
            jax.ShapeDtypeStruct((8, H), jnp.float32),
            jax.ShapeDtypeStruct((8, 128), jnp.int32),
            jax.ShapeDtypeStruct((8, H), jnp.float32),
            jax.ShapeDtypeStruct((8, H), jnp.float32),
        ],
        scratch_shapes=[pltpu.VMEM((8, H), jnp.float32)],
    )(x, W0, W1, W2, W3, bmat)

    rand = _replacement_rows()
    nw0, nw1, nw2, nw3 = pl.pallas_call(
        _weights_kernel,
        in_specs=[
            pl.BlockSpec(memory_space=pltpu.SMEM),
            pl.BlockSpec((8, H), lambda: (0, 0)),
            pl.BlockSpec(rand.shape, lambda: (0, 0)),
            pl.BlockSpec((H, H), lambda: (0, 0)),
            pl.BlockSpec((H, H), lambda: (0, 0)),
            pl.BlockSpec((H, H), lambda: (0, 0)),
            pl.BlockSpec((H, H), lambda: (0, 0)),
        ],
        out_specs=[pl.BlockSpec((H, H), lambda: (0, 0))] * 4,
        out_shape=[jax.ShapeDtypeStruct((H, H), jnp.float32)] * 4,
    )(featsp, keep8, rand, W0, W1, W2, W3)

    weight_util = wu8[:L]
    mean_act = ma8[:L]
    feats = featsp[:L, :K]
    newb0 = nb8[0]
    newb1 = nb8[1]
    newb2 = nb8[2]
    newb3 = nb8[3]
    return (weight_util, mean_act, feats,
            nw0, nw1, nw2, nw3, newb0, newb1, newb2, newb3)
